# Initial kernel scaffold; baseline (speedup 1.0000x reference)
import jax
import jax.numpy as jnp
from jax.experimental import pallas as pl


def kernel(atom_features, bond_features, pair_indices, kernel, bias):
    raise NotImplementedError("write your pallas kernel here")


# speedup vs baseline: 5.3135x; 5.3135x over previous
import functools

import jax
import jax.numpy as jnp
from jax import lax
from jax.experimental import pallas as pl
from jax.experimental.pallas import tpu as pltpu
from jax.experimental.pallas import tpu_sc as plsc

ATOM = 32
BOND = 16
N_NODES = 10000
N_EDGES = 160000

EDGE_BLK = 2000

_NC, _NS = 2, 16
_NW = _NC * _NS
_E_PER_W = N_EDGES // _NW
_CHUNK = 1000
_ROWS_PER_S = N_NODES // _NS


def _sc_gather(atom_features, idx_dst):
    mesh = plsc.VectorSubcoreMesh(core_axis_name="c", subcore_axis_name="s")

    @functools.partial(
        pl.kernel,
        out_type=jax.ShapeDtypeStruct((N_EDGES, ATOM), jnp.float32),
        mesh=mesh,
        scratch_types=[
            pltpu.VMEM((_CHUNK,), jnp.int32),
            pltpu.VMEM((_CHUNK, ATOM), jnp.float32),
            pltpu.SemaphoreType.DMA,
        ],
        compiler_params=pltpu.CompilerParams(use_tc_tiling_on_sc=False),
    )
    def gather_kernel(table_hbm, idx_hbm, out_hbm, idx_v, rows_v, sem):
        wid = lax.axis_index("s") * _NC + lax.axis_index("c")
        base = wid * _E_PER_W

        def chunk(i, carry):
            off = base + i * _CHUNK
            pltpu.sync_copy(idx_hbm.at[pl.ds(off, _CHUNK)], idx_v)
            pltpu.async_copy(table_hbm.at[idx_v], rows_v, sem).wait()
            pltpu.sync_copy(rows_v, out_hbm.at[pl.ds(off, _CHUNK)])
            return carry

        lax.fori_loop(0, _E_PER_W // _CHUNK, chunk, 0)

    return gather_kernel(atom_features, idx_dst)


def _sc_scatter(t, idx_src, zeros_nodes):
    mesh = plsc.VectorSubcoreMesh(core_axis_name="c", subcore_axis_name="s")

    @functools.partial(
        pl.kernel,
        out_type=jax.ShapeDtypeStruct((_NC, N_NODES, ATOM), jnp.float32),
        mesh=mesh,
        scratch_types=[
            pltpu.VMEM((_CHUNK,), jnp.int32),
            pltpu.VMEM((_CHUNK, ATOM), jnp.float32),
            pltpu.VMEM_SHARED((N_NODES, ATOM), jnp.float32),
            pltpu.SemaphoreType.DMA,
        ],
        compiler_params=pltpu.CompilerParams(use_tc_tiling_on_sc=False),
    )
    def scatter_kernel(t_hbm, idx_hbm, zeros_hbm, out_hbm,
                       idx_v, rows_v, acc_sh, sem):
        c = lax.axis_index("c")
        s = lax.axis_index("s")
        wid = s * _NC + c
        pltpu.sync_copy(zeros_hbm.at[pl.ds(s * _ROWS_PER_S, _ROWS_PER_S)],
                        acc_sh.at[pl.ds(s * _ROWS_PER_S, _ROWS_PER_S)])
        plsc.subcore_barrier()

        base = wid * _E_PER_W

        def chunk(i, carry):
            off = base + i * _CHUNK
            pltpu.sync_copy(idx_hbm.at[pl.ds(off, _CHUNK)], idx_v)
            pltpu.sync_copy(t_hbm.at[pl.ds(off, _CHUNK)], rows_v)
            pltpu.sync_copy(rows_v, acc_sh.at[idx_v], add=True)
            return carry

        lax.fori_loop(0, _E_PER_W // _CHUNK, chunk, 0)
        plsc.subcore_barrier()
        pltpu.sync_copy(acc_sh.at[pl.ds(s * _ROWS_PER_S, _ROWS_PER_S)],
                        out_hbm.at[c, pl.ds(s * _ROWS_PER_S, _ROWS_PER_S)])

    return scatter_kernel(t, idx_src, zeros_nodes)


def _edge_transform_body(bond_ref, x_ref, w_ref, out_ref):
    xt = x_ref[...].T
    yt = jnp.dot(w_ref[...], xt,
                 preferred_element_type=jnp.float32)
    bt = bond_ref[...].T
    acc = yt[BOND * ATOM:(BOND + 1) * ATOM, :]
    for b in range(BOND):
        acc = acc + yt[b * ATOM:(b + 1) * ATOM, :] * bt[b:b + 1, :]
    out_ref[...] = acc.T


def _edge_transform(bond_features, x, w):
    grid = (N_EDGES // EDGE_BLK,)
    return pl.pallas_call(
        _edge_transform_body,
        grid=grid,
        in_specs=[
            pl.BlockSpec((EDGE_BLK, BOND), lambda i: (i, 0)),
            pl.BlockSpec((EDGE_BLK, ATOM), lambda i: (i, 0)),
            pl.BlockSpec(((BOND + 1) * ATOM, ATOM), lambda i: (0, 0)),
        ],
        out_specs=pl.BlockSpec((EDGE_BLK, ATOM), lambda i: (i, 0)),
        out_shape=jax.ShapeDtypeStruct((N_EDGES, ATOM), jnp.float32),
    )(bond_features, x, w)


def _combine_body(p_ref, out_ref):
    out_ref[...] = p_ref[0] + p_ref[1]


def _combine(partials):
    return pl.pallas_call(
        _combine_body,
        out_shape=jax.ShapeDtypeStruct((N_NODES, ATOM), jnp.float32),
    )(partials)


def kernel(atom_features, bond_features, pair_indices, kernel, bias):
    idx_dst = pair_indices[:, 1]
    idx_src = pair_indices[:, 0]
    k_aug = jnp.concatenate([kernel, bias[None, :]], axis=0)
    w = k_aug.reshape((BOND + 1) * ATOM, ATOM)
    zeros_nodes = jnp.zeros((N_NODES, ATOM), jnp.float32)

    x = _sc_gather(atom_features, idx_dst)
    t = _edge_transform(bond_features, x, w)
    partials = _sc_scatter(t, idx_src, zeros_nodes)
    return _combine(partials)


# speedup vs baseline: 5.4868x; 1.0326x over previous
import functools

import jax
import jax.numpy as jnp
from jax import lax
from jax.experimental import pallas as pl
from jax.experimental.pallas import tpu as pltpu
from jax.experimental.pallas import tpu_sc as plsc

ATOM = 32
LANE = 128
BOND = 16
N_NODES = 10000
N_EDGES = 160000

EDGE_BLK = 2000

_NC, _NS = 2, 16
_NW = _NC * _NS
_E_PER_W = N_EDGES // _NW
_CHUNK = 1000
_ROWS_PER_S = N_NODES // _NS


def _sc_gather(atom128, idx_dst):
    mesh = plsc.VectorSubcoreMesh(core_axis_name="c", subcore_axis_name="s")

    @functools.partial(
        pl.kernel,
        out_type=jax.ShapeDtypeStruct((N_EDGES, LANE), jnp.float32),
        mesh=mesh,
        scratch_types=[
            pltpu.VMEM((_CHUNK,), jnp.int32),
            pltpu.VMEM((_CHUNK, LANE), jnp.float32),
            pltpu.SemaphoreType.DMA,
        ],
    )
    def gather_kernel(table_hbm, idx_hbm, out_hbm, idx_v, rows_v, sem):
        wid = lax.axis_index("s") * _NC + lax.axis_index("c")
        base = wid * _E_PER_W

        def chunk(i, carry):
            off = base + i * _CHUNK
            pltpu.sync_copy(idx_hbm.at[pl.ds(off, _CHUNK)], idx_v)
            pltpu.async_copy(table_hbm.at[idx_v], rows_v, sem).wait()
            pltpu.sync_copy(rows_v, out_hbm.at[pl.ds(off, _CHUNK)])
            return carry

        lax.fori_loop(0, _E_PER_W // _CHUNK, chunk, 0)

    return gather_kernel(atom128, idx_dst)


def _sc_scatter(t, idx_src, zeros_nodes):
    mesh = plsc.VectorSubcoreMesh(core_axis_name="c", subcore_axis_name="s")

    @functools.partial(
        pl.kernel,
        out_type=jax.ShapeDtypeStruct((_NC, N_NODES, ATOM), jnp.float32),
        mesh=mesh,
        scratch_types=[
            pltpu.VMEM((_CHUNK,), jnp.int32),
            pltpu.VMEM((_CHUNK, ATOM), jnp.float32),
            pltpu.VMEM_SHARED((N_NODES, ATOM), jnp.float32),
            pltpu.SemaphoreType.DMA,
        ],
        compiler_params=pltpu.CompilerParams(use_tc_tiling_on_sc=False),
    )
    def scatter_kernel(t_hbm, idx_hbm, zeros_hbm, out_hbm,
                       idx_v, rows_v, acc_sh, sem):
        c = lax.axis_index("c")
        s = lax.axis_index("s")
        wid = s * _NC + c
        pltpu.sync_copy(zeros_hbm.at[pl.ds(s * _ROWS_PER_S, _ROWS_PER_S)],
                        acc_sh.at[pl.ds(s * _ROWS_PER_S, _ROWS_PER_S)])
        plsc.subcore_barrier()

        base = wid * _E_PER_W

        def chunk(i, carry):
            off = base + i * _CHUNK
            pltpu.sync_copy(idx_hbm.at[pl.ds(off, _CHUNK)], idx_v)
            pltpu.sync_copy(t_hbm.at[pl.ds(off, _CHUNK)], rows_v)
            pltpu.sync_copy(rows_v, acc_sh.at[idx_v], add=True)
            return carry

        lax.fori_loop(0, _E_PER_W // _CHUNK, chunk, 0)
        plsc.subcore_barrier()
        pltpu.sync_copy(acc_sh.at[pl.ds(s * _ROWS_PER_S, _ROWS_PER_S)],
                        out_hbm.at[c, pl.ds(s * _ROWS_PER_S, _ROWS_PER_S)])

    return scatter_kernel(t, idx_src, zeros_nodes)


def _edge_transform_body(bond_ref, x_ref, w_ref, out_ref):
    xt = x_ref[...].T
    yt = jnp.dot(w_ref[...], xt,
                 preferred_element_type=jnp.float32)
    bt = bond_ref[...].T
    acc = yt[BOND * ATOM:(BOND + 1) * ATOM, :]
    for b in range(BOND):
        acc = acc + yt[b * ATOM:(b + 1) * ATOM, :] * bt[b:b + 1, :]
    out_ref[...] = acc.T


def _edge_transform(bond_features, x128, w128):
    grid = (N_EDGES // EDGE_BLK,)
    return pl.pallas_call(
        _edge_transform_body,
        grid=grid,
        in_specs=[
            pl.BlockSpec((EDGE_BLK, BOND), lambda i: (i, 0)),
            pl.BlockSpec((EDGE_BLK, LANE), lambda i: (i, 0)),
            pl.BlockSpec(((BOND + 1) * ATOM, LANE), lambda i: (0, 0)),
        ],
        out_specs=pl.BlockSpec((EDGE_BLK, ATOM), lambda i: (i, 0)),
        out_shape=jax.ShapeDtypeStruct((N_EDGES, ATOM), jnp.float32),
    )(bond_features, x128, w128)


def _combine_body(p_ref, out_ref):
    out_ref[...] = p_ref[0] + p_ref[1]


def _combine(partials):
    return pl.pallas_call(
        _combine_body,
        out_shape=jax.ShapeDtypeStruct((N_NODES, ATOM), jnp.float32),
    )(partials)


def kernel(atom_features, bond_features, pair_indices, kernel, bias):
    idx_dst = pair_indices[:, 1]
    idx_src = pair_indices[:, 0]
    k_aug = jnp.concatenate([kernel, bias[None, :]], axis=0)
    w = k_aug.reshape((BOND + 1) * ATOM, ATOM)
    w128 = jnp.pad(w, ((0, 0), (0, LANE - ATOM)))
    atom128 = jnp.pad(atom_features, ((0, 0), (0, LANE - ATOM)))
    zeros_nodes = jnp.zeros((N_NODES, ATOM), jnp.float32)

    x128 = _sc_gather(atom128, idx_dst)
    t = _edge_transform(bond_features, x128, w128)
    partials = _sc_scatter(t, idx_src, zeros_nodes)
    return _combine(partials)


# speedup vs baseline: 6.8582x; 1.2499x over previous
import functools

import jax
import jax.numpy as jnp
from jax import lax
from jax.experimental import pallas as pl
from jax.experimental.pallas import tpu as pltpu
from jax.experimental.pallas import tpu_sc as plsc

ATOM = 32
LANE = 128
PACK = LANE // ATOM
BOND = 16
N_NODES = 10000
N_EDGES = 160000
E_PAD = 163840
E4 = E_PAD // PACK
N_ACC = 10016

EDGE_BLK4 = 512

_NC, _NS = 2, 16
_NW = _NC * _NS
_E_PER_W = E_PAD // _NW
_CHUNK = 1024
_ROWS_PER_S = N_ACC // _NS


def _sc_gather(atom_features, idx_dst):
    mesh = plsc.VectorSubcoreMesh(core_axis_name="c", subcore_axis_name="s")

    @functools.partial(
        pl.kernel,
        out_type=jax.ShapeDtypeStruct((E_PAD, ATOM), jnp.float32),
        mesh=mesh,
        scratch_types=[
            pltpu.VMEM((_CHUNK,), jnp.int32),
            pltpu.VMEM((_CHUNK, ATOM), jnp.float32),
            pltpu.SemaphoreType.DMA,
        ],
        compiler_params=pltpu.CompilerParams(use_tc_tiling_on_sc=False),
    )
    def gather_kernel(table_hbm, idx_hbm, out_hbm, idx_v, rows_v, sem):
        wid = lax.axis_index("s") * _NC + lax.axis_index("c")
        base = wid * _E_PER_W

        def chunk(i, carry):
            off = base + i * _CHUNK
            pltpu.sync_copy(idx_hbm.at[pl.ds(off, _CHUNK)], idx_v)
            pltpu.async_copy(table_hbm.at[idx_v], rows_v, sem).wait()
            pltpu.sync_copy(rows_v, out_hbm.at[pl.ds(off, _CHUNK)])
            return carry

        lax.fori_loop(0, _E_PER_W // _CHUNK, chunk, 0)

    return gather_kernel(atom_features, idx_dst)


def _sc_scatter(t, idx_src, zeros_acc):
    mesh = plsc.VectorSubcoreMesh(core_axis_name="c", subcore_axis_name="s")

    @functools.partial(
        pl.kernel,
        out_type=jax.ShapeDtypeStruct((_NC, N_ACC, ATOM), jnp.float32),
        mesh=mesh,
        scratch_types=[
            pltpu.VMEM((_CHUNK,), jnp.int32),
            pltpu.VMEM((_CHUNK, ATOM), jnp.float32),
            pltpu.VMEM_SHARED((N_ACC, ATOM), jnp.float32),
            pltpu.SemaphoreType.DMA,
        ],
        compiler_params=pltpu.CompilerParams(use_tc_tiling_on_sc=False),
    )
    def scatter_kernel(t_hbm, idx_hbm, zeros_hbm, out_hbm,
                       idx_v, rows_v, acc_sh, sem):
        c = lax.axis_index("c")
        s = lax.axis_index("s")
        wid = s * _NC + c
        pltpu.sync_copy(zeros_hbm.at[pl.ds(s * _ROWS_PER_S, _ROWS_PER_S)],
                        acc_sh.at[pl.ds(s * _ROWS_PER_S, _ROWS_PER_S)])
        plsc.subcore_barrier()

        base = wid * _E_PER_W

        def chunk(i, carry):
            off = base + i * _CHUNK
            pltpu.sync_copy(idx_hbm.at[pl.ds(off, _CHUNK)], idx_v)
            pltpu.sync_copy(t_hbm.at[pl.ds(off, _CHUNK)], rows_v)
            pltpu.sync_copy(rows_v, acc_sh.at[idx_v], add=True)
            return carry

        lax.fori_loop(0, _E_PER_W // _CHUNK, chunk, 0)
        plsc.subcore_barrier()
        pltpu.sync_copy(acc_sh.at[pl.ds(s * _ROWS_PER_S, _ROWS_PER_S)],
                        out_hbm.at[c, pl.ds(s * _ROWS_PER_S, _ROWS_PER_S)])

    return scatter_kernel(t, idx_src, zeros_acc)


def _edge_transform_body(bond4t_ref, x4_ref, w4_ref, out_ref):
    x4t = x4_ref[...].T
    yt = jnp.dot(w4_ref[...], x4t,
                 preferred_element_type=jnp.float32)
    btt = bond4t_ref[...]
    accs = []
    for k in range(PACK):
        o = k * (BOND + 1) * ATOM
        acc = yt[o + BOND * ATOM:o + (BOND + 1) * ATOM, :]
        for b in range(BOND):
            acc = acc + (yt[o + b * ATOM:o + (b + 1) * ATOM, :]
                         * btt[k * BOND + b:k * BOND + b + 1, :])
        accs.append(acc)
    out_ref[...] = jnp.concatenate(accs, axis=0).T


def _edge_transform(bond4t, x4, w4):
    grid = (E4 // EDGE_BLK4,)
    return pl.pallas_call(
        _edge_transform_body,
        grid=grid,
        in_specs=[
            pl.BlockSpec((PACK * BOND, EDGE_BLK4), lambda i: (0, i)),
            pl.BlockSpec((EDGE_BLK4, LANE), lambda i: (i, 0)),
            pl.BlockSpec((PACK * (BOND + 1) * ATOM, LANE), lambda i: (0, 0)),
        ],
        out_specs=pl.BlockSpec((EDGE_BLK4, LANE), lambda i: (i, 0)),
        out_shape=jax.ShapeDtypeStruct((E4, LANE), jnp.float32),
    )(bond4t, x4, w4)


def _combine_body(p_ref, out_ref):
    out_ref[...] = p_ref[0, :N_NODES, :] + p_ref[1, :N_NODES, :]


def _combine(partials):
    return pl.pallas_call(
        _combine_body,
        out_shape=jax.ShapeDtypeStruct((N_NODES, ATOM), jnp.float32),
    )(partials)


def kernel(atom_features, bond_features, pair_indices, kernel, bias):
    n_pad = E_PAD - N_EDGES
    idx_dst = jnp.pad(pair_indices[:, 1], (0, n_pad))
    idx_dst = idx_dst.reshape(PACK, E4).transpose(1, 0).reshape(E_PAD)
    idx_src = jnp.pad(pair_indices[:, 0], (0, n_pad),
                      constant_values=N_NODES)
    idx_src = idx_src.reshape(PACK, E4).transpose(1, 0).reshape(E_PAD)
    bond4t = jnp.pad(bond_features.T, ((0, 0), (0, n_pad)))
    bond4t = bond4t.reshape(BOND, PACK, E4).transpose(1, 0, 2)
    bond4t = bond4t.reshape(PACK * BOND, E4)
    k_aug = jnp.concatenate([kernel, bias[None, :]], axis=0)
    w = k_aug.reshape((BOND + 1) * ATOM, ATOM)
    w4 = jnp.kron(jnp.eye(PACK, dtype=jnp.float32), w)
    zeros_acc = jnp.zeros((N_ACC, ATOM), jnp.float32)

    x = _sc_gather(atom_features, idx_dst)
    x4 = x.reshape(E4, LANE)
    t4 = _edge_transform(bond4t, x4, w4)
    t = t4.reshape(E_PAD, ATOM)
    partials = _sc_scatter(t, idx_src, zeros_acc)
    return _combine(partials)


# speedup vs baseline: 7.1130x; 1.0372x over previous
import functools

import jax
import jax.numpy as jnp
from jax import lax
from jax.experimental import pallas as pl
from jax.experimental.pallas import tpu as pltpu
from jax.experimental.pallas import tpu_sc as plsc

ATOM = 32
LANE = 128
PACK = LANE // ATOM
BOND = 16
N_NODES = 10000
N_EDGES = 160000
E_PAD = 163840
E4 = E_PAD // PACK
N_ACC = 10016

EDGE_BLK4 = 1024

_NC, _NS = 2, 16
_NW = _NC * _NS
_E_PER_W = E_PAD // _NW
_CHUNK = 1024
_ROWS_PER_S = N_ACC // _NS


def _sc_gather(atom_features, idx_dst):
    mesh = plsc.VectorSubcoreMesh(core_axis_name="c", subcore_axis_name="s")

    @functools.partial(
        pl.kernel,
        out_type=jax.ShapeDtypeStruct((E_PAD, ATOM), jnp.float32),
        mesh=mesh,
        scratch_types=[
            pltpu.VMEM((_CHUNK,), jnp.int32),
            pltpu.VMEM((_CHUNK, ATOM), jnp.float32),
            pltpu.SemaphoreType.DMA,
        ],
        compiler_params=pltpu.CompilerParams(use_tc_tiling_on_sc=False),
    )
    def gather_kernel(table_hbm, idx_hbm, out_hbm, idx_v, rows_v, sem):
        wid = lax.axis_index("s") * _NC + lax.axis_index("c")
        base = wid * _E_PER_W

        def chunk(i, carry):
            off = base + i * _CHUNK
            pltpu.sync_copy(idx_hbm.at[pl.ds(off, _CHUNK)], idx_v)
            pltpu.async_copy(table_hbm.at[idx_v], rows_v, sem).wait()
            pltpu.sync_copy(rows_v, out_hbm.at[pl.ds(off, _CHUNK)])
            return carry

        lax.fori_loop(0, _E_PER_W // _CHUNK, chunk, 0)

    return gather_kernel(atom_features, idx_dst)


def _sc_scatter(t, idx_src, zeros_acc):
    mesh = plsc.VectorSubcoreMesh(core_axis_name="c", subcore_axis_name="s")

    @functools.partial(
        pl.kernel,
        out_type=jax.ShapeDtypeStruct((_NC, N_ACC, ATOM), jnp.float32),
        mesh=mesh,
        scratch_types=[
            pltpu.VMEM((_CHUNK,), jnp.int32),
            pltpu.VMEM((_CHUNK, ATOM), jnp.float32),
            pltpu.VMEM_SHARED((N_ACC, ATOM), jnp.float32),
            pltpu.SemaphoreType.DMA,
        ],
        compiler_params=pltpu.CompilerParams(use_tc_tiling_on_sc=False),
    )
    def scatter_kernel(t_hbm, idx_hbm, zeros_hbm, out_hbm,
                       idx_v, rows_v, acc_sh, sem):
        c = lax.axis_index("c")
        s = lax.axis_index("s")
        wid = s * _NC + c
        pltpu.sync_copy(zeros_hbm.at[pl.ds(s * _ROWS_PER_S, _ROWS_PER_S)],
                        acc_sh.at[pl.ds(s * _ROWS_PER_S, _ROWS_PER_S)])
        plsc.subcore_barrier()

        base = wid * _E_PER_W

        def chunk(i, carry):
            off = base + i * _CHUNK
            pltpu.sync_copy(idx_hbm.at[pl.ds(off, _CHUNK)], idx_v)
            pltpu.sync_copy(t_hbm.at[pl.ds(off, _CHUNK)], rows_v)
            pltpu.sync_copy(rows_v, acc_sh.at[idx_v], add=True)
            return carry

        lax.fori_loop(0, _E_PER_W // _CHUNK, chunk, 0)
        plsc.subcore_barrier()
        pltpu.sync_copy(acc_sh.at[pl.ds(s * _ROWS_PER_S, _ROWS_PER_S)],
                        out_hbm.at[c, pl.ds(s * _ROWS_PER_S, _ROWS_PER_S)])

    return scatter_kernel(t, idx_src, zeros_acc)


def _edge_transform_body(bond4t_ref, x4_ref, w4_ref, out_ref):
    x4t = x4_ref[...].T
    yt = jnp.dot(w4_ref[...], x4t,
                 preferred_element_type=jnp.float32)
    btt = bond4t_ref[...]
    accs = []
    for k in range(PACK):
        o = k * (BOND + 1) * ATOM
        acc = yt[o + BOND * ATOM:o + (BOND + 1) * ATOM, :]
        for b in range(BOND):
            acc = acc + (yt[o + b * ATOM:o + (b + 1) * ATOM, :]
                         * btt[k * BOND + b:k * BOND + b + 1, :])
        accs.append(acc)
    out_ref[...] = jnp.concatenate(accs, axis=0).T


def _edge_transform(bond4t, x4, w4):
    grid = (E4 // EDGE_BLK4,)
    return pl.pallas_call(
        _edge_transform_body,
        grid=grid,
        in_specs=[
            pl.BlockSpec((PACK * BOND, EDGE_BLK4), lambda i: (0, i)),
            pl.BlockSpec((EDGE_BLK4, LANE), lambda i: (i, 0)),
            pl.BlockSpec((PACK * (BOND + 1) * ATOM, LANE), lambda i: (0, 0)),
        ],
        out_specs=pl.BlockSpec((EDGE_BLK4, LANE), lambda i: (i, 0)),
        out_shape=jax.ShapeDtypeStruct((E4, LANE), jnp.float32),
    )(bond4t, x4, w4)


def _combine_body(p_ref, out_ref):
    out_ref[...] = p_ref[0, :N_NODES, :] + p_ref[1, :N_NODES, :]


def _combine(partials):
    return pl.pallas_call(
        _combine_body,
        out_shape=jax.ShapeDtypeStruct((N_NODES, ATOM), jnp.float32),
    )(partials)


def kernel(atom_features, bond_features, pair_indices, kernel, bias):
    n_pad = E_PAD - N_EDGES
    e = jnp.arange(E_PAD, dtype=jnp.int32)
    perm = (e % PACK) * E4 + e // PACK
    idx_dst = jnp.take(jnp.pad(pair_indices[:, 1], (0, n_pad)), perm)
    idx_src = jnp.take(jnp.pad(pair_indices[:, 0], (0, n_pad),
                               constant_values=N_NODES), perm)
    bond4t = jnp.pad(bond_features.T, ((0, 0), (0, n_pad)))
    bond4t = bond4t.reshape(BOND, PACK, E4).transpose(1, 0, 2)
    bond4t = bond4t.reshape(PACK * BOND, E4)
    k_aug = jnp.concatenate([kernel, bias[None, :]], axis=0)
    w = k_aug.reshape((BOND + 1) * ATOM, ATOM)
    w4 = jnp.kron(jnp.eye(PACK, dtype=jnp.float32), w)
    zeros_acc = jnp.zeros((N_ACC, ATOM), jnp.float32)

    x = _sc_gather(atom_features, idx_dst)
    x4 = x.reshape(E4, LANE)
    t4 = _edge_transform(bond4t, x4, w4)
    t = t4.reshape(E_PAD, ATOM)
    partials = _sc_scatter(t, idx_src, zeros_acc)
    return _combine(partials)


# speedup vs baseline: 8.7578x; 1.2312x over previous
import functools

import jax
import jax.numpy as jnp
from jax import lax
from jax.experimental import pallas as pl
from jax.experimental.pallas import tpu as pltpu
from jax.experimental.pallas import tpu_sc as plsc

ATOM = 32
LANE = 128
PACK = LANE // ATOM
BOND = 16
N_NODES = 10000
N_EDGES = 160000
E_PAD = 163840
E4 = E_PAD // PACK
N_HALF = 2
E_H = E_PAD // N_HALF
E4_H = E4 // N_HALF
N_ACC = 10016

EDGE_BLK4 = 1024

_NC, _NS = 2, 16
_NW = _NC * _NS
_S_PER_W = E_H // _NW
_CHUNK = 1280
_CH4 = _CHUNK // PACK
_ROWS_PER_S = N_ACC // _NS


def _load_idx_slot_order(idx_hbm, idx_stage, idx_v, r_abs, sem):
    for k in range(PACK):
        pltpu.sync_copy(idx_hbm.at[pl.ds(k * E4 + r_abs, _CH4)],
                        idx_stage.at[pl.ds(k * _CH4, _CH4)])
    lane = lax.iota(jnp.int32, 16)
    for g in range(PACK * (_CH4 // 16)):
        k, rr_base = g // (_CH4 // 16), (g % (_CH4 // 16)) * 16
        vals = idx_stage[pl.ds(g * 16, 16)]
        pos = lane * PACK + (PACK * rr_base + k)
        plsc.store_scatter(idx_v, [pos], vals)


def _sc_gather(atom_features, idx_dst, half):
    mesh = plsc.VectorSubcoreMesh(core_axis_name="c", subcore_axis_name="s")

    @functools.partial(
        pl.kernel,
        out_type=jax.ShapeDtypeStruct((E_H, ATOM), jnp.float32),
        mesh=mesh,
        scratch_types=[
            pltpu.VMEM((_CHUNK,), jnp.int32),
            pltpu.VMEM((_CHUNK,), jnp.int32),
            pltpu.VMEM((_CHUNK, ATOM), jnp.float32),
            pltpu.SemaphoreType.DMA,
        ],
        compiler_params=pltpu.CompilerParams(use_tc_tiling_on_sc=False,
                                             needs_layout_passes=False),
    )
    def gather_kernel(table_hbm, idx_hbm, out_hbm, idx_stage, idx_v,
                      rows_v, sem):
        wid = lax.axis_index("s") * _NC + lax.axis_index("c")

        def chunk(i, carry):
            off = wid * _S_PER_W + i * _CHUNK
            r_abs = half * E4_H + wid * (_S_PER_W // PACK) + i * _CH4
            _load_idx_slot_order(idx_hbm, idx_stage, idx_v, r_abs, sem)
            pltpu.async_copy(table_hbm.at[idx_v], rows_v, sem).wait()
            pltpu.sync_copy(rows_v, out_hbm.at[pl.ds(off, _CHUNK)])
            return carry

        lax.fori_loop(0, _S_PER_W // _CHUNK, chunk, 0)

    return gather_kernel(atom_features, idx_dst)


def _sc_scatter(t_h, idx_src, zeros_acc, half):
    mesh = plsc.VectorSubcoreMesh(core_axis_name="c", subcore_axis_name="s")

    @functools.partial(
        pl.kernel,
        out_type=jax.ShapeDtypeStruct((_NC, N_ACC, ATOM), jnp.float32),
        mesh=mesh,
        scratch_types=[
            pltpu.VMEM((_CHUNK,), jnp.int32),
            pltpu.VMEM((_CHUNK,), jnp.int32),
            pltpu.VMEM((_CHUNK, ATOM), jnp.float32),
            pltpu.VMEM_SHARED((N_ACC, ATOM), jnp.float32),
            pltpu.SemaphoreType.DMA,
        ],
        compiler_params=pltpu.CompilerParams(use_tc_tiling_on_sc=False,
                                             needs_layout_passes=False),
    )
    def scatter_kernel(t_hbm, idx_hbm, zeros_hbm, out_hbm,
                       idx_stage, idx_v, rows_v, acc_sh, sem):
        c = lax.axis_index("c")
        s = lax.axis_index("s")
        wid = s * _NC + c
        pltpu.sync_copy(zeros_hbm.at[pl.ds(s * _ROWS_PER_S, _ROWS_PER_S)],
                        acc_sh.at[pl.ds(s * _ROWS_PER_S, _ROWS_PER_S)])
        plsc.subcore_barrier()

        def chunk(i, carry):
            off = wid * _S_PER_W + i * _CHUNK
            r_abs = half * E4_H + wid * (_S_PER_W // PACK) + i * _CH4
            _load_idx_slot_order(idx_hbm, idx_stage, idx_v, r_abs, sem)
            pltpu.sync_copy(t_hbm.at[pl.ds(off, _CHUNK)], rows_v)
            pltpu.sync_copy(rows_v, acc_sh.at[idx_v], add=True)
            return carry

        lax.fori_loop(0, _S_PER_W // _CHUNK, chunk, 0)
        plsc.subcore_barrier()
        pltpu.sync_copy(acc_sh.at[pl.ds(s * _ROWS_PER_S, _ROWS_PER_S)],
                        out_hbm.at[c, pl.ds(s * _ROWS_PER_S, _ROWS_PER_S)])

    return scatter_kernel(t_h, idx_src, zeros_acc)


def _edge_transform_body(bt0_ref, bt1_ref, bt2_ref, bt3_ref, x4_ref, w4_ref,
                         out_ref):
    x4t = x4_ref[...].T
    yt = jnp.dot(w4_ref[...], x4t,
                 preferred_element_type=jnp.float32)
    btt = jnp.concatenate([bt0_ref[...], bt1_ref[...],
                           bt2_ref[...], bt3_ref[...]], axis=0)
    accs = []
    for k in range(PACK):
        o = k * (BOND + 1) * ATOM
        acc = yt[o + BOND * ATOM:o + (BOND + 1) * ATOM, :]
        for b in range(BOND):
            acc = acc + (yt[o + b * ATOM:o + (b + 1) * ATOM, :]
                         * btt[k * BOND + b:k * BOND + b + 1, :])
        accs.append(acc)
    out_ref[...] = jnp.concatenate(accs, axis=0).T


def _edge_transform(bond_t, x4_h, w4, half):
    grid = (E4_H // EDGE_BLK4,)
    nb = EDGE_BLK4
    bond_specs = [
        pl.BlockSpec((BOND, EDGE_BLK4),
                     functools.partial(
                         lambda k, i: (0, (k * E4 + half * E4_H) // nb + i),
                         k))
        for k in range(PACK)
    ]
    return pl.pallas_call(
        _edge_transform_body,
        grid=grid,
        in_specs=bond_specs + [
            pl.BlockSpec((EDGE_BLK4, LANE), lambda i: (i, 0)),
            pl.BlockSpec((PACK * (BOND + 1) * ATOM, LANE), lambda i: (0, 0)),
        ],
        out_specs=pl.BlockSpec((EDGE_BLK4, LANE), lambda i: (i, 0)),
        out_shape=jax.ShapeDtypeStruct((E4_H, LANE), jnp.float32),
    )(bond_t, bond_t, bond_t, bond_t, x4_h, w4)


def _combine_body(p0_ref, p1_ref, out_ref):
    s = (p0_ref[0] + p0_ref[1]) + (p1_ref[0] + p1_ref[1])
    out_ref[...] = s[:N_NODES, :]


def _combine(partials0, partials1):
    return pl.pallas_call(
        _combine_body,
        out_shape=jax.ShapeDtypeStruct((N_NODES, ATOM), jnp.float32),
    )(partials0, partials1)


def kernel(atom_features, bond_features, pair_indices, kernel, bias):
    n_pad = E_PAD - N_EDGES
    idx_dst = jnp.pad(pair_indices[:, 1], (0, n_pad))
    idx_src = jnp.pad(pair_indices[:, 0], (0, n_pad),
                      constant_values=N_NODES)
    bond_t = jnp.pad(bond_features.T, ((0, 0), (0, n_pad)))
    k_aug = jnp.concatenate([kernel, bias[None, :]], axis=0)
    w = k_aug.reshape((BOND + 1) * ATOM, ATOM)
    w4 = jnp.kron(jnp.eye(PACK, dtype=jnp.float32), w)
    zeros_acc = jnp.zeros((N_ACC, ATOM), jnp.float32)

    partials = []
    for h in range(N_HALF):
        x_h = _sc_gather(atom_features, idx_dst, h)
        x4_h = x_h.reshape(E4_H, LANE)
        t4_h = _edge_transform(bond_t, x4_h, w4, h)
        t_h = t4_h.reshape(E_H, ATOM)
        partials.append(_sc_scatter(t_h, idx_src, zeros_acc, h))
    return _combine(*partials)


# speedup vs baseline: 10.5607x; 1.2059x over previous
import functools

import jax
import jax.numpy as jnp
from jax import lax
from jax.experimental import pallas as pl
from jax.experimental.pallas import tpu as pltpu
from jax.experimental.pallas import tpu_sc as plsc

ATOM = 32
LANE = 128
PACK = LANE // ATOM
BOND = 16
N_NODES = 10000
N_EDGES = 160000
E_PAD = 163840
E4 = E_PAD // PACK
N_Q = 4
E_Q = E_PAD // N_Q
E4_Q = E4 // N_Q
N_ACC = 10016

EDGE_BLK4 = 1024

_NC, _NS = 2, 16
_NW = _NC * _NS
_S_PER_W = E_Q // _NW
_CH4 = _S_PER_W // PACK
_ROWS_PER_S = N_ACC // _NS
_PAR_ROWS = _NC * N_ACC * ATOM // LANE


def _load_idx_slot_order(pair_hbm, row, idx_stage, idx_v, r_abs, sem):
    copies = [
        pltpu.async_copy(pair_hbm.at[row, pl.ds(k * E4 + r_abs, _CH4)],
                         idx_stage.at[pl.ds(k * _CH4, _CH4)], sem)
        for k in range(PACK)
    ]
    for cp in copies:
        cp.wait()
    lane = lax.iota(jnp.int32, 16)
    for g in range(PACK * (_CH4 // 16)):
        k, rr_base = g // (_CH4 // 16), (g % (_CH4 // 16)) * 16
        vals = idx_stage[pl.ds(g * 16, 16)]
        pos = lane * PACK + (PACK * rr_base + k)
        plsc.store_scatter(idx_v, [pos], vals)


def _sc_gather(atom_features, pair_t, quarter):
    mesh = plsc.VectorSubcoreMesh(core_axis_name="c", subcore_axis_name="s")

    @functools.partial(
        pl.kernel,
        out_type=jax.ShapeDtypeStruct((E_Q, ATOM), jnp.float32),
        mesh=mesh,
        scratch_types=[
            pltpu.VMEM((_S_PER_W,), jnp.int32),
            pltpu.VMEM((_S_PER_W,), jnp.int32),
            pltpu.VMEM((_S_PER_W, ATOM), jnp.float32),
            pltpu.SemaphoreType.DMA,
        ],
        compiler_params=pltpu.CompilerParams(use_tc_tiling_on_sc=False,
                                             needs_layout_passes=False),
    )
    def gather_kernel(table_hbm, pair_hbm, out_hbm, idx_stage, idx_v,
                      rows_v, sem):
        wid = lax.axis_index("s") * _NC + lax.axis_index("c")
        off = wid * _S_PER_W
        r_abs = quarter * E4_Q + wid * _CH4
        _load_idx_slot_order(pair_hbm, 1, idx_stage, idx_v, r_abs, sem)
        pltpu.async_copy(table_hbm.at[idx_v], rows_v, sem).wait()
        pltpu.sync_copy(rows_v, out_hbm.at[pl.ds(off, _S_PER_W)])

    return gather_kernel(atom_features, pair_t)


def _sc_scatter(t_q, pair_t, zeros_acc, quarter):
    mesh = plsc.VectorSubcoreMesh(core_axis_name="c", subcore_axis_name="s")

    @functools.partial(
        pl.kernel,
        out_type=jax.ShapeDtypeStruct((_NC, N_ACC, ATOM), jnp.float32),
        mesh=mesh,
        scratch_types=[
            pltpu.VMEM((_S_PER_W,), jnp.int32),
            pltpu.VMEM((_S_PER_W,), jnp.int32),
            pltpu.VMEM((_S_PER_W, ATOM), jnp.float32),
            pltpu.VMEM_SHARED((N_ACC, ATOM), jnp.float32),
            pltpu.SemaphoreType.DMA,
        ],
        compiler_params=pltpu.CompilerParams(use_tc_tiling_on_sc=False,
                                             needs_layout_passes=False),
    )
    def scatter_kernel(t_hbm, pair_hbm, zeros_hbm, out_hbm,
                       idx_stage, idx_v, rows_v, acc_sh, sem):
        c = lax.axis_index("c")
        s = lax.axis_index("s")
        wid = s * _NC + c
        pltpu.sync_copy(zeros_hbm.at[pl.ds(s * _ROWS_PER_S, _ROWS_PER_S)],
                        acc_sh.at[pl.ds(s * _ROWS_PER_S, _ROWS_PER_S)])
        off = wid * _S_PER_W
        r_abs = quarter * E4_Q + wid * _CH4
        _load_idx_slot_order(pair_hbm, 0, idx_stage, idx_v, r_abs, sem)
        pltpu.sync_copy(t_hbm.at[pl.ds(off, _S_PER_W)], rows_v)
        plsc.subcore_barrier()
        pltpu.sync_copy(rows_v, acc_sh.at[idx_v], add=True)
        plsc.subcore_barrier()
        pltpu.sync_copy(acc_sh.at[pl.ds(s * _ROWS_PER_S, _ROWS_PER_S)],
                        out_hbm.at[c, pl.ds(s * _ROWS_PER_S, _ROWS_PER_S)])

    return scatter_kernel(t_q, pair_t, zeros_acc)


def _edge_transform_body(bt0_ref, bt1_ref, bt2_ref, bt3_ref, x4_ref, w4_ref,
                         out_ref):
    x4t = x4_ref[...].T
    yt = jnp.dot(w4_ref[...], x4t,
                 preferred_element_type=jnp.float32)
    btt = jnp.concatenate([bt0_ref[...], bt1_ref[...],
                           bt2_ref[...], bt3_ref[...]], axis=0)
    accs = []
    for k in range(PACK):
        o = k * (BOND + 1) * ATOM
        acc = yt[o + BOND * ATOM:o + (BOND + 1) * ATOM, :]
        for b in range(BOND):
            acc = acc + (yt[o + b * ATOM:o + (b + 1) * ATOM, :]
                         * btt[k * BOND + b:k * BOND + b + 1, :])
        accs.append(acc)
    out_ref[...] = jnp.concatenate(accs, axis=0).T


def _edge_transform(bond_t, bond_k3, x4_q, w4, quarter):
    grid = (E4_Q // EDGE_BLK4,)
    nb = EDGE_BLK4
    bond_specs = [
        pl.BlockSpec((BOND, EDGE_BLK4),
                     functools.partial(
                         lambda k, i: (0, (k * E4 + quarter * E4_Q) // nb + i),
                         k))
        for k in range(PACK - 1)
    ] + [
        pl.BlockSpec((BOND, EDGE_BLK4),
                     lambda i: (0, quarter * E4_Q // nb + i)),
    ]
    return pl.pallas_call(
        _edge_transform_body,
        grid=grid,
        in_specs=bond_specs + [
            pl.BlockSpec((EDGE_BLK4, LANE), lambda i: (i, 0)),
            pl.BlockSpec((PACK * (BOND + 1) * ATOM, LANE), lambda i: (0, 0)),
        ],
        out_specs=pl.BlockSpec((EDGE_BLK4, LANE), lambda i: (i, 0)),
        out_shape=jax.ShapeDtypeStruct((E4_Q, LANE), jnp.float32),
    )(bond_t, bond_t, bond_t, bond_k3, x4_q, w4)


def _combine_body(p0_ref, p1_ref, p2_ref, p3_ref, out_ref):
    h = _PAR_ROWS // 2
    s = ((p0_ref[:h, :] + p0_ref[h:, :])
         + (p1_ref[:h, :] + p1_ref[h:, :])
         + (p2_ref[:h, :] + p2_ref[h:, :])
         + (p3_ref[:h, :] + p3_ref[h:, :]))
    out_ref[...] = s


def _combine(partials):
    packed = [p.reshape(_PAR_ROWS, LANE) for p in partials]
    summed = pl.pallas_call(
        _combine_body,
        out_shape=jax.ShapeDtypeStruct((_PAR_ROWS // 2, LANE), jnp.float32),
    )(*packed)
    return summed.reshape(N_ACC, ATOM)[:N_NODES]


def kernel(atom_features, bond_features, pair_indices, kernel, bias):
    n_pad = E_PAD - N_EDGES
    pad_cols = jnp.concatenate(
        [jnp.full((1, n_pad), N_NODES, jnp.int32),
         jnp.zeros((1, n_pad), jnp.int32)], axis=0)
    pair_t = jnp.concatenate([pair_indices.T, pad_cols], axis=1)
    bond_t = bond_features.T
    bond_k3 = jnp.pad(bond_t[:, (PACK - 1) * E4:], ((0, 0), (0, n_pad)))
    k_aug = jnp.concatenate([kernel, bias[None, :]], axis=0)
    w = k_aug.reshape((BOND + 1) * ATOM, ATOM)
    w4 = jnp.kron(jnp.eye(PACK, dtype=jnp.float32), w)
    zeros_acc = jnp.zeros((N_ACC, ATOM), jnp.float32)

    partials = []
    for q in range(N_Q):
        x_q = _sc_gather(atom_features, pair_t, q)
        x4_q = x_q.reshape(E4_Q, LANE)
        t4_q = _edge_transform(bond_t, bond_k3, x4_q, w4, q)
        t_q = t4_q.reshape(E_Q, ATOM)
        partials.append(_sc_scatter(t_q, pair_t, zeros_acc, q))
    return _combine(partials)


# speedup vs baseline: 10.6114x; 1.0048x over previous
import functools

import jax
import jax.numpy as jnp
from jax import lax
from jax.experimental import pallas as pl
from jax.experimental.pallas import tpu as pltpu
from jax.experimental.pallas import tpu_sc as plsc

ATOM = 32
LANE = 128
PACK = LANE // ATOM
BOND = 16
N_NODES = 10000
N_EDGES = 160000
E_PAD = 163840
E4 = E_PAD // PACK
N_Q = 4
E_Q = E_PAD // N_Q
E4_Q = E4 // N_Q
N_ACC = 10016

EDGE_BLK4 = 1024

_NC, _NS = 2, 16
_NW = _NC * _NS
_S_PER_W = E_Q // _NW
_CH4 = _S_PER_W // PACK
_ROWS_PER_S = N_ACC // _NS
_PAR_ROWS = _NC * N_ACC * ATOM // LANE


def _load_idx_slot_order(pair_hbm, row, idx_stage, idx_v, r_abs, sem):
    copies = [
        pltpu.async_copy(pair_hbm.at[row, pl.ds(k * E4 + r_abs, _CH4)],
                         idx_stage.at[pl.ds(k * _CH4, _CH4)], sem)
        for k in range(PACK)
    ]
    for cp in copies:
        cp.wait()
    lane = lax.iota(jnp.int32, 16)
    for g in range(PACK * (_CH4 // 16)):
        k, rr_base = g // (_CH4 // 16), (g % (_CH4 // 16)) * 16
        vals = idx_stage[pl.ds(g * 16, 16)]
        pos = lane * PACK + (PACK * rr_base + k)
        plsc.store_scatter(idx_v, [pos], vals)


def _sc_gather(atom_features, pair_t, quarter):
    mesh = plsc.VectorSubcoreMesh(core_axis_name="c", subcore_axis_name="s")

    @functools.partial(
        pl.kernel,
        out_type=jax.ShapeDtypeStruct((E_Q, ATOM), jnp.float32),
        mesh=mesh,
        scratch_types=[
            pltpu.VMEM((_S_PER_W,), jnp.int32),
            pltpu.VMEM((_S_PER_W,), jnp.int32),
            pltpu.VMEM((_S_PER_W, ATOM), jnp.float32),
            pltpu.SemaphoreType.DMA,
        ],
        compiler_params=pltpu.CompilerParams(use_tc_tiling_on_sc=False,
                                             needs_layout_passes=False),
    )
    def gather_kernel(table_hbm, pair_hbm, out_hbm, idx_stage, idx_v,
                      rows_v, sem):
        wid = lax.axis_index("s") * _NC + lax.axis_index("c")
        off = wid * _S_PER_W
        r_abs = quarter * E4_Q + wid * _CH4
        _load_idx_slot_order(pair_hbm, 1, idx_stage, idx_v, r_abs, sem)
        pltpu.async_copy(table_hbm.at[idx_v], rows_v, sem).wait()
        pltpu.sync_copy(rows_v, out_hbm.at[pl.ds(off, _S_PER_W)])

    return gather_kernel(atom_features, pair_t)


def _sc_scatter(t_q, pair_t, zeros_acc, quarter):
    mesh = plsc.VectorSubcoreMesh(core_axis_name="c", subcore_axis_name="s")

    @functools.partial(
        pl.kernel,
        out_type=jax.ShapeDtypeStruct((_NC, N_ACC, ATOM), jnp.float32),
        mesh=mesh,
        scratch_types=[
            pltpu.VMEM((_S_PER_W,), jnp.int32),
            pltpu.VMEM((_S_PER_W,), jnp.int32),
            pltpu.VMEM((_S_PER_W, ATOM), jnp.float32),
            pltpu.VMEM_SHARED((N_ACC, ATOM), jnp.float32),
            pltpu.SemaphoreType.DMA,
        ],
        compiler_params=pltpu.CompilerParams(use_tc_tiling_on_sc=False,
                                             needs_layout_passes=False),
    )
    def scatter_kernel(t_hbm, pair_hbm, zeros_hbm, out_hbm,
                       idx_stage, idx_v, rows_v, acc_sh, sem):
        c = lax.axis_index("c")
        s = lax.axis_index("s")
        wid = s * _NC + c
        pltpu.sync_copy(zeros_hbm.at[pl.ds(s * _ROWS_PER_S, _ROWS_PER_S)],
                        acc_sh.at[pl.ds(s * _ROWS_PER_S, _ROWS_PER_S)])
        off = wid * _S_PER_W
        r_abs = quarter * E4_Q + wid * _CH4
        _load_idx_slot_order(pair_hbm, 0, idx_stage, idx_v, r_abs, sem)
        pltpu.sync_copy(t_hbm.at[pl.ds(off, _S_PER_W)], rows_v)
        plsc.subcore_barrier()
        pltpu.sync_copy(rows_v, acc_sh.at[idx_v], add=True)
        plsc.subcore_barrier()
        pltpu.sync_copy(acc_sh.at[pl.ds(s * _ROWS_PER_S, _ROWS_PER_S)],
                        out_hbm.at[c, pl.ds(s * _ROWS_PER_S, _ROWS_PER_S)])

    return scatter_kernel(t_q, pair_t, zeros_acc)


def _edge_transform_body(bt0_ref, bt1_ref, bt2_ref, bt3_ref, x4_ref, w4_ref,
                         out_ref):
    x4t = x4_ref[...].astype(jnp.bfloat16).T
    yt = jnp.dot(w4_ref[...], x4t,
                 preferred_element_type=jnp.float32)
    btt = jnp.concatenate([bt0_ref[...], bt1_ref[...],
                           bt2_ref[...], bt3_ref[...]], axis=0)
    accs = []
    for k in range(PACK):
        o = k * (BOND + 1) * ATOM
        acc = yt[o + BOND * ATOM:o + (BOND + 1) * ATOM, :]
        for b in range(BOND):
            acc = acc + (yt[o + b * ATOM:o + (b + 1) * ATOM, :]
                         * btt[k * BOND + b:k * BOND + b + 1, :])
        accs.append(acc)
    out_ref[...] = jnp.concatenate(accs, axis=0).T


def _edge_transform(bond_t, bond_k3, x4_q, w4, quarter):
    grid = (E4_Q // EDGE_BLK4,)
    nb = EDGE_BLK4
    bond_specs = [
        pl.BlockSpec((BOND, EDGE_BLK4),
                     functools.partial(
                         lambda k, i: (0, (k * E4 + quarter * E4_Q) // nb + i),
                         k))
        for k in range(PACK - 1)
    ] + [
        pl.BlockSpec((BOND, EDGE_BLK4),
                     lambda i: (0, quarter * E4_Q // nb + i)),
    ]
    return pl.pallas_call(
        _edge_transform_body,
        grid=grid,
        in_specs=bond_specs + [
            pl.BlockSpec((EDGE_BLK4, LANE), lambda i: (i, 0)),
            pl.BlockSpec((PACK * (BOND + 1) * ATOM, LANE), lambda i: (0, 0)),
        ],
        out_specs=pl.BlockSpec((EDGE_BLK4, LANE), lambda i: (i, 0)),
        out_shape=jax.ShapeDtypeStruct((E4_Q, LANE), jnp.float32),
    )(bond_t, bond_t, bond_t, bond_k3, x4_q, w4)


def _combine_body(p0_ref, p1_ref, p2_ref, p3_ref, out_ref):
    h = _PAR_ROWS // 2
    s = ((p0_ref[:h, :] + p0_ref[h:, :])
         + (p1_ref[:h, :] + p1_ref[h:, :])
         + (p2_ref[:h, :] + p2_ref[h:, :])
         + (p3_ref[:h, :] + p3_ref[h:, :]))
    out_ref[...] = s


def _combine(partials):
    packed = [p.reshape(_PAR_ROWS, LANE) for p in partials]
    summed = pl.pallas_call(
        _combine_body,
        out_shape=jax.ShapeDtypeStruct((_PAR_ROWS // 2, LANE), jnp.float32),
    )(*packed)
    return summed.reshape(N_ACC, ATOM)[:N_NODES]


def kernel(atom_features, bond_features, pair_indices, kernel, bias):
    n_pad = E_PAD - N_EDGES
    pad_cols = jnp.concatenate(
        [jnp.full((1, n_pad), N_NODES, jnp.int32),
         jnp.zeros((1, n_pad), jnp.int32)], axis=0)
    pair_t = jnp.concatenate([pair_indices.T, pad_cols], axis=1)
    bond_t = bond_features.T
    bond_k3 = jnp.pad(bond_t[:, (PACK - 1) * E4:], ((0, 0), (0, n_pad)))
    k_aug = jnp.concatenate([kernel, bias[None, :]], axis=0)
    w = k_aug.reshape((BOND + 1) * ATOM, ATOM)
    w4 = jnp.kron(jnp.eye(PACK, dtype=jnp.float32), w)
    w4 = w4.astype(jnp.bfloat16)
    zeros_acc = jnp.zeros((N_ACC, ATOM), jnp.float32)

    partials = []
    for q in range(N_Q):
        x_q = _sc_gather(atom_features, pair_t, q)
        x4_q = x_q.reshape(E4_Q, LANE)
        t4_q = _edge_transform(bond_t, bond_k3, x4_q, w4, q)
        t_q = t4_q.reshape(E_Q, ATOM)
        partials.append(_sc_scatter(t_q, pair_t, zeros_acc, q))
    return _combine(partials)
